# Initial kernel scaffold; baseline (speedup 1.0000x reference)
import jax
import jax.numpy as jnp
from jax.experimental import pallas as pl


def kernel(year, month, day, weekday, stores, items, emb_month, emb_day, emb_weekday, emb_stores, emb_items, W1, b1, W2, b2, W3, b3):
    raise NotImplementedError("write your pallas kernel here")


# speedup vs baseline: 1.4790x; 1.4790x over previous
import jax
import jax.numpy as jnp
from jax import lax
from jax.experimental import pallas as pl
from jax.experimental.pallas import tpu as pltpu
from jax.experimental.pallas import tpu_sc as plsc

_NC = 2
_NS = 16
_NW = _NC * _NS
_CH = 128


def _sc_gather_body(tm, td, tw, ts, ti, im, id_, iw, is_, ii,
                    om, od, ow, os_, oi,
                    vm, vd, vw, vs, vi, bm, bd, bw, bs, bi, sem):
  wid = lax.axis_index("s") * _NC + lax.axis_index("c")
  nch = im.shape[0] // _NW
  rows = nch * _CH

  idx_hbm = (im, id_, iw, is_, ii)
  idx_v = (vm, vd, vw, vs, vi)
  tables = (tm, td, tw, ts, ti)
  bufs = (bm, bd, bw, bs, bi)
  outs = (om, od, ow, os_, oi)

  for f in range(5):
    pltpu.sync_copy(idx_hbm[f].at[pl.ds(wid * nch, nch)], idx_v[f])

  cps = []
  for f in range(5):
    for j in range(nch):
      cps.append(pltpu.async_copy(
          tables[f].at[idx_v[f].at[j]],
          bufs[f].at[pl.ds(j * _CH, _CH)],
          sem))
  for cp in cps:
    cp.wait()

  base = wid * rows
  for f in range(5):
    pltpu.sync_copy(bufs[f], outs[f].at[pl.ds(base, rows)])


def _sc_gather(tables, idxs, batch):
  rows_per_w = batch // _NW
  nch = rows_per_w // _CH
  kern = pl.kernel(
      _sc_gather_body,
      out_type=(
          jax.ShapeDtypeStruct((batch, 16), jnp.float32),
          jax.ShapeDtypeStruct((batch, 16), jnp.float32),
          jax.ShapeDtypeStruct((batch, 16), jnp.float32),
          jax.ShapeDtypeStruct((batch, 16), jnp.float32),
          jax.ShapeDtypeStruct((batch, 32), jnp.float32),
      ),
      mesh=plsc.VectorSubcoreMesh(core_axis_name="c", subcore_axis_name="s"),
      compiler_params=pltpu.CompilerParams(use_tc_tiling_on_sc=False),
      scratch_types=[
          pltpu.VMEM((nch, _CH), jnp.int32),
          pltpu.VMEM((nch, _CH), jnp.int32),
          pltpu.VMEM((nch, _CH), jnp.int32),
          pltpu.VMEM((nch, _CH), jnp.int32),
          pltpu.VMEM((nch, _CH), jnp.int32),
          pltpu.VMEM((rows_per_w, 16), jnp.float32),
          pltpu.VMEM((rows_per_w, 16), jnp.float32),
          pltpu.VMEM((rows_per_w, 16), jnp.float32),
          pltpu.VMEM((rows_per_w, 16), jnp.float32),
          pltpu.VMEM((rows_per_w, 32), jnp.float32),
          pltpu.SemaphoreType.DMA,
      ],
  )
  return kern(*tables, *idxs)


def _tc_mlp_body(xm_ref, xd_ref, xw_ref, xs_ref, xi_ref, y_ref,
                 w1_ref, b1_ref, w2_ref, b2_ref, w3_ref, b3_ref, o_ref):
  w1 = w1_ref[...]

  def z(n):
    return jnp.zeros((n, w1.shape[1]), jnp.float32)

  def mm(a, b):
    return jnp.dot(a, b, preferred_element_type=jnp.float32,
                   precision=lax.Precision.HIGHEST)

  h = (mm(xm_ref[...], jnp.concatenate([w1[1:8], z(9)], axis=0))
       + mm(xd_ref[...], w1[8:24])
       + mm(xw_ref[...], jnp.concatenate([w1[24:28], z(12)], axis=0))
       + mm(xs_ref[...], jnp.concatenate([w1[28:34], z(10)], axis=0))
       + mm(xi_ref[...], jnp.concatenate([w1[34:60], z(6)], axis=0)))
  h = jnp.maximum(h + y_ref[...] * w1[0:1, :] + b1_ref[...], 0.0)
  h = jnp.maximum(mm(h, w2_ref[...]) + b2_ref[...], 0.0)
  o_ref[...] = mm(h, w3_ref[...]) + b3_ref[...]


def _tc_mlp(xs, year, w1, b1, w2, b2, w3, b3, blk=2048):
  batch = year.shape[0]
  grid = batch // blk
  full = lambda a: pl.BlockSpec(a.shape, lambda i: (0,) * a.ndim)
  return pl.pallas_call(
      _tc_mlp_body,
      grid=(grid,),
      in_specs=[pl.BlockSpec((blk, x.shape[1]), lambda i: (i, 0)) for x in xs]
      + [pl.BlockSpec((blk, 1), lambda i: (i, 0)),
         full(w1), full(b1), full(w2), full(b2), full(w3), full(b3)],
      out_specs=pl.BlockSpec((blk, 1), lambda i: (i, 0)),
      out_shape=jax.ShapeDtypeStruct((batch, 1), jnp.float32),
  )(*xs, year, w1, b1, w2, b2, w3, b3)


def kernel(year, month, day, weekday, stores, items,
           emb_month, emb_day, emb_weekday, emb_stores, emb_items,
           W1, b1, W2, b2, W3, b3):
  batch = year.shape[0]
  nch_total = batch // _CH

  pad = lambda t, w: jnp.pad(t, ((0, 0), (0, w - t.shape[1])))
  tables = (pad(emb_month, 16), pad(emb_day, 16), pad(emb_weekday, 16),
            pad(emb_stores, 16), pad(emb_items, 32))
  idxs = tuple(a.reshape(nch_total, _CH)
               for a in (month, day, weekday, stores, items))

  xs = _sc_gather(tables, idxs, batch)
  return _tc_mlp(xs, year, W1, b1.reshape(1, -1), W2, b2.reshape(1, -1),
                 W3, b3.reshape(1, -1))


# speedup vs baseline: 4.8142x; 3.2551x over previous
import jax
import jax.numpy as jnp
from jax import lax
from jax.experimental import pallas as pl
from jax.experimental.pallas import tpu as pltpu
from jax.experimental.pallas import tpu_sc as plsc

_NC = 2
_NS = 16
_NW = _NC * _NS
_L = 16

_TBL_ROWS = 115
_TBL_W = 32
_WIDTHS = (7, 16, 4, 6, 26)
_XROWS = 59


def _sc_gather_body(tbl_hbm, idx_hbm, xt_hbm, tbl_v, idx_v, buf_v):
  wid = lax.axis_index("s") * _NC + lax.axis_index("c")
  rows = idx_hbm.shape[1] // _NW
  base = wid * rows

  pltpu.sync_copy(tbl_hbm, tbl_v)
  pltpu.sync_copy(idx_hbm.at[:, pl.ds(base, rows)], idx_v)

  def block(rb, _):
    off = rb * _L
    g = 0
    for f in range(5):
      iv = idx_v[f, pl.ds(off, _L)] * _TBL_W
      for c in range(_WIDTHS[f]):
        v = plsc.load_gather(tbl_v, [iv + c])
        buf_v[g, pl.ds(off, _L)] = v
        g += 1
    return _

  lax.fori_loop(0, rows // _L, block, None)
  pltpu.sync_copy(buf_v, xt_hbm.at[:, pl.ds(base, rows)])


def _sc_gather(tbl, idx_all, batch):
  rows_per_w = batch // _NW
  kern = pl.kernel(
      _sc_gather_body,
      out_type=jax.ShapeDtypeStruct((_XROWS, batch), jnp.float32),
      mesh=plsc.VectorSubcoreMesh(core_axis_name="c", subcore_axis_name="s"),
      compiler_params=pltpu.CompilerParams(use_tc_tiling_on_sc=False,
                                           needs_layout_passes=False),
      scratch_types=[
          pltpu.VMEM((_TBL_ROWS * _TBL_W,), jnp.float32),
          pltpu.VMEM((5, rows_per_w), jnp.int32),
          pltpu.VMEM((_XROWS, rows_per_w), jnp.float32),
      ],
  )
  return kern(tbl, idx_all)


def _tc_mlp_body(xt_ref, yt_ref, w1_ref, b1_ref, w2_ref, b2_ref, w3_ref,
                 b3_ref, o_ref):
  w1 = w1_ref[...]

  def mmT(a, b):
    return lax.dot_general(a, b, (((0,), (0,)), ((), ())),
                           preferred_element_type=jnp.float32,
                           precision=lax.Precision.HIGHEST)

  h = mmT(w1[1:60], xt_ref[...])
  h = h + mmT(w1[0:1], yt_ref[...]) + b1_ref[...]
  h = jnp.maximum(h, 0.0)
  h = jnp.maximum(mmT(w2_ref[...], h) + b2_ref[...], 0.0)
  o_ref[...] = mmT(w3_ref[...], h) + b3_ref[...]


def _tc_mlp(xt, yt, w1, b1, w2, b2, w3, b3, blk=2048):
  batch = yt.shape[1]
  grid = batch // blk
  full = lambda a: pl.BlockSpec(a.shape, lambda i: (0,) * a.ndim)
  return pl.pallas_call(
      _tc_mlp_body,
      grid=(grid,),
      in_specs=[
          pl.BlockSpec((_XROWS, blk), lambda i: (0, i)),
          pl.BlockSpec((1, blk), lambda i: (0, i)),
          full(w1), full(b1), full(w2), full(b2), full(w3), full(b3),
      ],
      out_specs=pl.BlockSpec((1, blk), lambda i: (0, i)),
      out_shape=jax.ShapeDtypeStruct((1, batch), jnp.float32),
  )(xt, yt, w1, b1, w2, b2, w3, b3)


def kernel(year, month, day, weekday, stores, items,
           emb_month, emb_day, emb_weekday, emb_stores, emb_items,
           W1, b1, W2, b2, W3, b3):
  batch = year.shape[0]

  pad = lambda t: jnp.pad(t, ((0, 0), (0, _TBL_W - t.shape[1])))
  tbl = jnp.concatenate(
      [pad(emb_month), pad(emb_day), pad(emb_weekday), pad(emb_stores),
       pad(emb_items)], axis=0).reshape(-1)
  idx_all = jnp.stack([
      month[:, 0], day[:, 0] + 13, weekday[:, 0] + 45, stores[:, 0] + 53,
      items[:, 0] + 64])

  xt = _sc_gather(tbl, idx_all, batch)
  out_t = _tc_mlp(xt, year.reshape(1, batch), W1, b1.reshape(-1, 1), W2,
                  b2.reshape(-1, 1), W3, b3.reshape(-1, 1))
  return out_t.reshape(batch, 1)


# speedup vs baseline: 6.4091x; 1.3313x over previous
import jax
import jax.numpy as jnp
from jax import lax
from jax.experimental import pallas as pl
from jax.experimental.pallas import tpu as pltpu
from jax.experimental.pallas import tpu_sc as plsc

_NC = 2
_NS = 16
_NW = _NC * _NS
_L = 16

_TBL_ROWS = 115
_TBL_W = 33
_WIDTHS = (7, 16, 4, 6, 26)
_XROWS = 59


def _sc_gather_body(tbl_hbm, idx_hbm, xt_hbm, tbl_v, idx_v, buf_v):
  wid = lax.axis_index("s") * _NC + lax.axis_index("c")
  rows = idx_hbm.shape[1] // _NW
  base = wid * rows

  pltpu.sync_copy(tbl_hbm, tbl_v)
  pltpu.sync_copy(idx_hbm.at[:, pl.ds(base, rows)], idx_v)

  @plsc.parallel_loop(0, rows // _L, unroll=2)
  def block(rb):
    off = rb * _L
    g = 0
    for f in range(5):
      iv = idx_v[f, pl.ds(off, _L)] * _TBL_W
      for c in range(_WIDTHS[f]):
        v = plsc.load_gather(tbl_v, [iv + c])
        buf_v[g, pl.ds(off, _L)] = v
        g += 1
  pltpu.sync_copy(buf_v, xt_hbm.at[:, pl.ds(base, rows)])


def _sc_gather(tbl, idx_all, batch):
  rows_per_w = batch // _NW
  kern = pl.kernel(
      _sc_gather_body,
      out_type=jax.ShapeDtypeStruct((_XROWS, batch), jnp.float32),
      mesh=plsc.VectorSubcoreMesh(core_axis_name="c", subcore_axis_name="s"),
      compiler_params=pltpu.CompilerParams(use_tc_tiling_on_sc=False,
                                           needs_layout_passes=False),
      scratch_types=[
          pltpu.VMEM((_TBL_ROWS * _TBL_W,), jnp.float32),
          pltpu.VMEM((5, rows_per_w), jnp.int32),
          pltpu.VMEM((_XROWS, rows_per_w), jnp.float32),
      ],
  )
  return kern(tbl, idx_all)


def _tc_mlp_body(xt_ref, yt_ref, w1_ref, b1_ref, w2_ref, b2_ref, w3_ref,
                 b3_ref, o_ref):
  w1 = w1_ref[...]

  def mmT(a, b):
    return lax.dot_general(a, b, (((0,), (0,)), ((), ())),
                           preferred_element_type=jnp.float32,
                           precision=lax.Precision.HIGHEST)

  h = mmT(w1[1:60], xt_ref[...])
  h = h + mmT(w1[0:1], yt_ref[...]) + b1_ref[...]
  h = jnp.maximum(h, 0.0)
  h = jnp.maximum(mmT(w2_ref[...], h) + b2_ref[...], 0.0)
  o_ref[...] = mmT(w3_ref[...], h) + b3_ref[...]


def _tc_mlp(xt, yt, w1, b1, w2, b2, w3, b3, blk=2048):
  batch = yt.shape[1]
  grid = batch // blk
  full = lambda a: pl.BlockSpec(a.shape, lambda i: (0,) * a.ndim)
  return pl.pallas_call(
      _tc_mlp_body,
      grid=(grid,),
      in_specs=[
          pl.BlockSpec((_XROWS, blk), lambda i: (0, i)),
          pl.BlockSpec((1, blk), lambda i: (0, i)),
          full(w1), full(b1), full(w2), full(b2), full(w3), full(b3),
      ],
      out_specs=pl.BlockSpec((1, blk), lambda i: (0, i)),
      out_shape=jax.ShapeDtypeStruct((1, batch), jnp.float32),
  )(xt, yt, w1, b1, w2, b2, w3, b3)


def kernel(year, month, day, weekday, stores, items,
           emb_month, emb_day, emb_weekday, emb_stores, emb_items,
           W1, b1, W2, b2, W3, b3):
  batch = year.shape[0]

  pad = lambda t: jnp.pad(t, ((0, 0), (0, _TBL_W - t.shape[1])))
  tbl = jnp.concatenate(
      [pad(emb_month), pad(emb_day), pad(emb_weekday), pad(emb_stores),
       pad(emb_items)], axis=0).reshape(-1)
  idx_all = jnp.stack([
      month[:, 0], day[:, 0] + 13, weekday[:, 0] + 45, stores[:, 0] + 53,
      items[:, 0] + 64])

  xt = _sc_gather(tbl, idx_all, batch)
  out_t = _tc_mlp(xt, year.reshape(1, batch), W1, b1.reshape(-1, 1), W2,
                  b2.reshape(-1, 1), W3, b3.reshape(-1, 1))
  return out_t.reshape(batch, 1)


# speedup vs baseline: 7.7983x; 1.2168x over previous
import jax
import jax.numpy as jnp
from jax import lax
from jax.experimental import pallas as pl
from jax.experimental.pallas import tpu as pltpu
from jax.experimental.pallas import tpu_sc as plsc

_NC = 2
_NS = 16
_NW = _NC * _NS
_L = 16

_TBL_ROWS = 115
_TBL_W = 33
_WIDTHS = (7, 16, 4, 6, 26)
_ROFF = (0, 13, 45, 53, 64)
_XROWS = 60


def _sc_gather_body(tbl_hbm, im, id_, iw, is_, ii, yr, xt_hbm,
                    tbl_v, idx_v, buf_v, sem):
  wid = lax.axis_index("s") * _NC + lax.axis_index("c")
  rows = yr.shape[0] // _NW
  base = wid * rows

  idxs = (im, id_, iw, is_, ii)

  cps = [pltpu.async_copy(tbl_hbm, tbl_v, sem)]
  for f in range(5):
    cps.append(pltpu.async_copy(
        idxs[f].at[pl.ds(base, rows)], idx_v.at[f], sem))
  cps.append(pltpu.async_copy(yr.at[pl.ds(base, rows)], buf_v.at[59], sem))
  for cp in cps:
    cp.wait()

  @plsc.parallel_loop(0, rows // _L, unroll=2)
  def block(rb):
    off = rb * _L
    g = 0
    for f in range(5):
      iv = idx_v[f, pl.ds(off, _L)] * _TBL_W + _ROFF[f] * _TBL_W
      for c in range(_WIDTHS[f]):
        v = plsc.load_gather(tbl_v, [iv + c])
        buf_v[g, pl.ds(off, _L)] = v
        g += 1

  pltpu.sync_copy(buf_v, xt_hbm.at[:, pl.ds(base, rows)])


def _sc_gather(tbl, idxs, yr, batch):
  rows_per_w = batch // _NW
  kern = pl.kernel(
      _sc_gather_body,
      out_type=jax.ShapeDtypeStruct((_XROWS, batch), jnp.float32),
      mesh=plsc.VectorSubcoreMesh(core_axis_name="c", subcore_axis_name="s"),
      compiler_params=pltpu.CompilerParams(use_tc_tiling_on_sc=False,
                                           needs_layout_passes=False),
      scratch_types=[
          pltpu.VMEM((_TBL_ROWS * _TBL_W,), jnp.float32),
          pltpu.VMEM((5, rows_per_w), jnp.int32),
          pltpu.VMEM((_XROWS, rows_per_w), jnp.float32),
          pltpu.SemaphoreType.DMA,
      ],
  )
  return kern(tbl, *idxs, yr)


def _tc_mlp_body(xt_ref, w1_ref, b1_ref, w2_ref, b2_ref, w3_ref, b3_ref,
                 o_ref):
  def mmT(a, b):
    return lax.dot_general(a, b, (((0,), (0,)), ((), ())),
                           preferred_element_type=jnp.float32,
                           precision=lax.Precision.HIGHEST)

  w1 = w1_ref[...]
  w1r = jnp.concatenate([w1[1:60], w1[0:1]], axis=0)
  h = jnp.maximum(mmT(w1r, xt_ref[...]) + b1_ref[...], 0.0)
  h = jnp.maximum(mmT(w2_ref[...], h) + b2_ref[...], 0.0)
  o_ref[...] = mmT(w3_ref[...], h) + b3_ref[...]


def _tc_mlp(xt, w1, b1, w2, b2, w3, b3, blk=4096):
  batch = xt.shape[1]
  grid = batch // blk
  full = lambda a: pl.BlockSpec(a.shape, lambda i: (0,) * a.ndim)
  return pl.pallas_call(
      _tc_mlp_body,
      grid=(grid,),
      in_specs=[
          pl.BlockSpec((_XROWS, blk), lambda i: (0, i)),
          full(w1), full(b1), full(w2), full(b2), full(w3), full(b3),
      ],
      out_specs=pl.BlockSpec((1, blk), lambda i: (0, i)),
      out_shape=jax.ShapeDtypeStruct((1, batch), jnp.float32),
  )(xt, w1, b1, w2, b2, w3, b3)


def kernel(year, month, day, weekday, stores, items,
           emb_month, emb_day, emb_weekday, emb_stores, emb_items,
           W1, b1, W2, b2, W3, b3):
  batch = year.shape[0]
  pad = lambda t: jnp.pad(t, ((0, 0), (0, _TBL_W - t.shape[1])))
  tbl = jnp.concatenate(
      [pad(emb_month), pad(emb_day), pad(emb_weekday), pad(emb_stores),
       pad(emb_items)], axis=0).reshape(-1)
  idxs = tuple(a.reshape(batch) for a in (month, day, weekday, stores, items))

  xt = _sc_gather(tbl, idxs, year.reshape(batch), batch)
  out_t = _tc_mlp(xt, W1, b1.reshape(-1, 1), W2, b2.reshape(-1, 1), W3,
                  b3.reshape(-1, 1))
  return out_t.reshape(batch, 1)
